# Initial kernel scaffold; baseline (speedup 1.0000x reference)
import jax
import jax.numpy as jnp
from jax.experimental import pallas as pl


def kernel(sparse_inputs, embedding_inputs, w):
    raise NotImplementedError("write your pallas kernel here")


# speedup vs baseline: 1.0658x; 1.0658x over previous
import functools

import jax
import jax.numpy as jnp
from jax import lax
from jax.experimental import pallas as pl
from jax.experimental.pallas import tpu as pltpu
from jax.experimental.pallas import tpu_sc as plsc

BATCH = 16384
N_FIELDS = 26
EMBED_DIM = 16
COLS = N_FIELDS * EMBED_DIM

NC, NS, LANES = 2, 16, 16
NW = NC * NS
B_PER_W = BATCH // NW
IDX_PER_W = B_PER_W * N_FIELDS


def _first_order_sc(idx_r, w_flat):
    mesh = plsc.VectorSubcoreMesh(core_axis_name="c", subcore_axis_name="s")

    @functools.partial(
        pl.kernel,
        out_type=jax.ShapeDtypeStruct((BATCH,), jnp.float32),
        mesh=mesh,
        scratch_types=[
            pltpu.VMEM((IDX_PER_W,), jnp.int32),
            pltpu.VMEM((IDX_PER_W,), jnp.float32),
            pltpu.VMEM((B_PER_W,), jnp.float32),
            pltpu.SemaphoreType.DMA,
        ],
    )
    def k(idx_hbm, w_hbm, out_hbm, idx_v, vals_v, out_v, sem):
        wid = lax.axis_index("s") * NC + lax.axis_index("c")
        pltpu.sync_copy(idx_hbm.at[pl.ds(wid * IDX_PER_W, IDX_PER_W)], idx_v)
        pltpu.async_copy(w_hbm.at[idx_v], vals_v, sem).wait()
        for jg in range(B_PER_W // LANES):
            acc = vals_v[pl.ds(jg * LANES, LANES)]
            for f in range(1, N_FIELDS):
                acc = acc + vals_v[pl.ds(f * B_PER_W + jg * LANES, LANES)]
            out_v[pl.ds(jg * LANES, LANES)] = acc
        pltpu.sync_copy(out_v, out_hbm.at[pl.ds(wid * B_PER_W, B_PER_W)])

    return k(idx_r, w_flat)


def _fm_tc(first, emb2d):
    BLK = 2048

    def body(first_ref, emb_ref, out_ref):
        e = emb_ref[...]
        col = lax.broadcasted_iota(jnp.int32, (COLS, EMBED_DIM), 0)
        dim = lax.broadcasted_iota(jnp.int32, (COLS, EMBED_DIM), 1)
        m = jnp.where(col % EMBED_DIM == dim, 1.0, 0.0)
        s = jnp.dot(e, m, preferred_element_type=jnp.float32)
        t1 = jnp.sum(s * s, axis=1, keepdims=True)
        t2 = jnp.sum(e * e, axis=1, keepdims=True)
        out_ref[...] = first_ref[...] + 0.5 * (t1 - t2)

    return pl.pallas_call(
        body,
        grid=(BATCH // BLK,),
        in_specs=[
            pl.BlockSpec((BLK, 1), lambda i: (i, 0)),
            pl.BlockSpec((BLK, COLS), lambda i: (i, 0)),
        ],
        out_specs=pl.BlockSpec((BLK, 1), lambda i: (i, 0)),
        out_shape=jax.ShapeDtypeStruct((BATCH, 1), jnp.float32),
    )(first, emb2d)


def kernel(sparse_inputs, embedding_inputs, w):
    idx_r = sparse_inputs.reshape(NW, B_PER_W, N_FIELDS).transpose(0, 2, 1).reshape(-1)
    w_flat = w.reshape(-1)
    first = _first_order_sc(idx_r, w_flat)
    emb2d = embedding_inputs.reshape(BATCH, COLS)
    return _fm_tc(first.reshape(BATCH, 1), emb2d)


# speedup vs baseline: 2.8758x; 2.6982x over previous
import functools

import jax
import jax.numpy as jnp
from jax import lax
from jax.experimental import pallas as pl
from jax.experimental.pallas import tpu as pltpu
from jax.experimental.pallas import tpu_sc as plsc

BATCH = 16384
N_FIELDS = 26
EMBED_DIM = 16

NC, NS, LANES = 2, 16, 16
NW = NC * NS
B_PER_W = BATCH // NW
IDX_PER_W = B_PER_W * N_FIELDS


def _first_order_sc(idx_r, w_row):
    mesh = plsc.VectorSubcoreMesh(core_axis_name="c", subcore_axis_name="s")

    @functools.partial(
        pl.kernel,
        out_type=jax.ShapeDtypeStruct((1, BATCH), jnp.float32),
        mesh=mesh,
        scratch_types=[
            pltpu.VMEM((IDX_PER_W,), jnp.int32),
            pltpu.VMEM((IDX_PER_W,), jnp.float32),
            pltpu.VMEM((B_PER_W,), jnp.float32),
            pltpu.SemaphoreType.DMA,
        ],
    )
    def k(idx_hbm, w_hbm, out_hbm, idx_v, vals_v, out_v, sem):
        wid = lax.axis_index("s") * NC + lax.axis_index("c")
        pltpu.sync_copy(idx_hbm.at[pl.ds(wid * IDX_PER_W, IDX_PER_W)], idx_v)
        pltpu.async_copy(w_hbm.at[0].at[idx_v], vals_v, sem).wait()
        for jg in range(B_PER_W // LANES):
            acc = vals_v[pl.ds(jg * LANES, LANES)]
            for f in range(1, N_FIELDS):
                acc = acc + vals_v[pl.ds(f * B_PER_W + jg * LANES, LANES)]
            out_v[pl.ds(jg * LANES, LANES)] = acc
        pltpu.sync_copy(out_v, out_hbm.at[0].at[pl.ds(wid * B_PER_W, B_PER_W)])

    return k(idx_r, w_row)


def _second_order_tc(et):
    BLK = 2048

    def body(e_ref, out_ref):
        x = e_ref[...]
        s = jnp.sum(x, axis=0)
        t1 = jnp.sum(s * s, axis=0, keepdims=True)
        t2 = jnp.sum(jnp.sum(x * x, axis=0), axis=0, keepdims=True)
        out_ref[...] = 0.5 * (t1 - t2)

    return pl.pallas_call(
        body,
        grid=(BATCH // BLK,),
        in_specs=[pl.BlockSpec((N_FIELDS, EMBED_DIM, BLK), lambda i: (0, 0, i))],
        out_specs=pl.BlockSpec((1, BLK), lambda i: (0, i)),
        out_shape=jax.ShapeDtypeStruct((1, BATCH), jnp.float32),
    )(et)


def _add_tc(a, b):
    def body(a_ref, b_ref, out_ref):
        out_ref[...] = a_ref[...] + b_ref[...]

    return pl.pallas_call(
        body,
        out_shape=jax.ShapeDtypeStruct((1, BATCH), jnp.float32),
    )(a, b)


def kernel(sparse_inputs, embedding_inputs, w):
    idx_r = (
        sparse_inputs.T.reshape(N_FIELDS, NW, B_PER_W)
        .transpose(1, 0, 2)
        .reshape(-1)
    )
    first = _first_order_sc(idx_r, w.T)
    second = _second_order_tc(embedding_inputs.transpose(1, 2, 0))
    return _add_tc(first, second).T


# speedup vs baseline: 2.9164x; 1.0141x over previous
import functools

import jax
import jax.numpy as jnp
from jax import lax
from jax.experimental import pallas as pl
from jax.experimental.pallas import tpu as pltpu
from jax.experimental.pallas import tpu_sc as plsc

BATCH = 16384
N_FIELDS = 26
EMBED_DIM = 16

NC, NS, LANES = 2, 16, 16
NW = NC * NS
B_PER_W = BATCH // NW
IDX_PER_W = B_PER_W * N_FIELDS


def _first_order_sc(idx_r, w_row):
    mesh = plsc.VectorSubcoreMesh(core_axis_name="c", subcore_axis_name="s")

    @functools.partial(
        pl.kernel,
        out_type=jax.ShapeDtypeStruct((1, BATCH), jnp.float32),
        mesh=mesh,
        scratch_types=[
            pltpu.VMEM((IDX_PER_W,), jnp.int32),
            pltpu.VMEM((IDX_PER_W,), jnp.float32),
            pltpu.VMEM((B_PER_W,), jnp.float32),
            pltpu.SemaphoreType.DMA,
        ],
    )
    def k(idx_hbm, w_hbm, out_hbm, idx_v, vals_v, out_v, sem):
        wid = lax.axis_index("s") * NC + lax.axis_index("c")
        pltpu.sync_copy(idx_hbm.at[pl.ds(wid * IDX_PER_W, IDX_PER_W)], idx_v)
        pltpu.async_copy(w_hbm.at[0].at[idx_v], vals_v, sem).wait()

        def reduce_group(jg, _):
            base = jg * LANES
            acc = vals_v[pl.ds(base, LANES)]
            for f in range(1, N_FIELDS):
                acc = acc + vals_v[pl.ds(f * B_PER_W + base, LANES)]
            out_v[pl.ds(base, LANES)] = acc
            return 0

        lax.fori_loop(0, B_PER_W // LANES, reduce_group, 0, unroll=False)
        pltpu.sync_copy(out_v, out_hbm.at[0].at[pl.ds(wid * B_PER_W, B_PER_W)])

    return k(idx_r, w_row)


def _second_order_tc(et):
    BLK = 2048

    def body(e_ref, out_ref):
        x = e_ref[...]
        s = jnp.sum(x, axis=0)
        t1 = jnp.sum(s * s, axis=0, keepdims=True)
        t2 = jnp.sum(jnp.sum(x * x, axis=0), axis=0, keepdims=True)
        out_ref[...] = 0.5 * (t1 - t2)

    return pl.pallas_call(
        body,
        grid=(BATCH // BLK,),
        in_specs=[pl.BlockSpec((N_FIELDS, EMBED_DIM, BLK), lambda i: (0, 0, i))],
        out_specs=pl.BlockSpec((1, BLK), lambda i: (0, i)),
        out_shape=jax.ShapeDtypeStruct((1, BATCH), jnp.float32),
    )(et)


def _add_tc(a, b):
    def body(a_ref, b_ref, out_ref):
        out_ref[...] = a_ref[...] + b_ref[...]

    return pl.pallas_call(
        body,
        out_shape=jax.ShapeDtypeStruct((1, BATCH), jnp.float32),
    )(a, b)


def kernel(sparse_inputs, embedding_inputs, w):
    idx_r = (
        sparse_inputs.T.reshape(N_FIELDS, NW, B_PER_W)
        .transpose(1, 0, 2)
        .reshape(-1)
    )
    first = _first_order_sc(idx_r, w.T)
    second = _second_order_tc(embedding_inputs.transpose(1, 2, 0))
    return _add_tc(first, second).T


# speedup vs baseline: 3.4537x; 1.1842x over previous
import functools

import jax
import jax.numpy as jnp
from jax import lax
from jax.experimental import pallas as pl
from jax.experimental.pallas import tpu as pltpu
from jax.experimental.pallas import tpu_sc as plsc

BATCH = 16384
N_FIELDS = 26
EMBED_DIM = 16

NC, NS, LANES = 2, 16, 16
NW = NC * NS
B_PER_W = BATCH // NW
IDX_PER_W = B_PER_W * N_FIELDS

FEAT = 1000000
PER_SUB = 62496
SCH = 10416
N_SCH = PER_SUB // SCH
REM = FEAT - NS * PER_SUB


def _first_order_sc(idx_r, w_row):
    mesh = plsc.VectorSubcoreMesh(core_axis_name="c", subcore_axis_name="s")

    @functools.partial(
        pl.kernel,
        out_type=jax.ShapeDtypeStruct((1, BATCH), jnp.float32),
        mesh=mesh,
        scratch_types=[
            pltpu.VMEM((IDX_PER_W,), jnp.int32),
            pltpu.VMEM((IDX_PER_W,), jnp.float32),
            pltpu.VMEM((B_PER_W,), jnp.float32),
            pltpu.VMEM((SCH,), jnp.float32),
            pltpu.VMEM((SCH,), jnp.float32),
            pltpu.VMEM_SHARED((FEAT,), jnp.float32),
            pltpu.SemaphoreType.DMA,
            pltpu.SemaphoreType.DMA,
            pltpu.SemaphoreType.DMA,
            pltpu.SemaphoreType.DMA,
            pltpu.SemaphoreType.DMA,
        ],
    )
    def k(idx_hbm, w_hbm, out_hbm, idx_v, vals_v, out_v, buf_a, buf_b,
          w_sh, sem, la, lb, sa, sb):
        sid = lax.axis_index("s")
        wid = sid * NC + lax.axis_index("c")
        pltpu.sync_copy(idx_hbm.at[pl.ds(wid * IDX_PER_W, IDX_PER_W)], idx_v)

        bufs = (buf_a, buf_b)
        lsems = (la, lb)
        ssems = (sa, sb)
        loads = [
            pltpu.make_async_copy(
                w_hbm.at[0].at[pl.ds(sid * PER_SUB + t * SCH, SCH)],
                bufs[t % 2], lsems[t % 2])
            for t in range(N_SCH)
        ]
        stores = [
            pltpu.make_async_copy(
                bufs[t % 2],
                w_sh.at[pl.ds(sid * PER_SUB + t * SCH, SCH)], ssems[t % 2])
            for t in range(N_SCH)
        ]
        loads[0].start()
        for t in range(N_SCH):
            if t + 1 < N_SCH:
                if t - 1 >= 0:
                    stores[t - 1].wait()
                loads[t + 1].start()
            loads[t].wait()
            stores[t].start()
        stores[N_SCH - 2].wait()
        stores[N_SCH - 1].wait()

        @pl.when(sid == 0)
        def _():
            pltpu.async_copy(w_hbm.at[0].at[pl.ds(NS * PER_SUB, REM)],
                             buf_a.at[pl.ds(0, REM)], la).wait()
            pltpu.async_copy(buf_a.at[pl.ds(0, REM)],
                             w_sh.at[pl.ds(NS * PER_SUB, REM)], sa).wait()

        plsc.subcore_barrier()
        pltpu.async_copy(w_sh.at[idx_v], vals_v, sem).wait()

        def reduce_group(jg, _):
            base = jg * LANES
            acc = vals_v[pl.ds(base, LANES)]
            for f in range(1, N_FIELDS):
                acc = acc + vals_v[pl.ds(f * B_PER_W + base, LANES)]
            out_v[pl.ds(base, LANES)] = acc
            return 0

        lax.fori_loop(0, B_PER_W // LANES, reduce_group, 0, unroll=False)
        pltpu.sync_copy(out_v, out_hbm.at[0].at[pl.ds(wid * B_PER_W, B_PER_W)])

    return k(idx_r, w_row)


def _second_order_tc(et):
    BLK = 2048

    def body(e_ref, out_ref):
        x = e_ref[...]
        s = jnp.sum(x, axis=0)
        t1 = jnp.sum(s * s, axis=0, keepdims=True)
        t2 = jnp.sum(jnp.sum(x * x, axis=0), axis=0, keepdims=True)
        out_ref[...] = 0.5 * (t1 - t2)

    return pl.pallas_call(
        body,
        grid=(BATCH // BLK,),
        in_specs=[pl.BlockSpec((N_FIELDS, EMBED_DIM, BLK), lambda i: (0, 0, i))],
        out_specs=pl.BlockSpec((1, BLK), lambda i: (0, i)),
        out_shape=jax.ShapeDtypeStruct((1, BATCH), jnp.float32),
    )(et)


def _add_tc(a, b):
    def body(a_ref, b_ref, out_ref):
        out_ref[...] = a_ref[...] + b_ref[...]

    return pl.pallas_call(
        body,
        out_shape=jax.ShapeDtypeStruct((1, BATCH), jnp.float32),
    )(a, b)


def kernel(sparse_inputs, embedding_inputs, w):
    idx_r = (
        sparse_inputs.T.reshape(N_FIELDS, NW, B_PER_W)
        .transpose(1, 0, 2)
        .reshape(-1)
    )
    first = _first_order_sc(idx_r, w.T)
    second = _second_order_tc(embedding_inputs.transpose(1, 2, 0))
    return _add_tc(first, second).T


# speedup vs baseline: 3.7524x; 1.0865x over previous
import functools

import jax
import jax.numpy as jnp
from jax import lax
from jax.experimental import pallas as pl
from jax.experimental.pallas import tpu as pltpu
from jax.experimental.pallas import tpu_sc as plsc

BATCH = 16384
N_FIELDS = 26
EMBED_DIM = 16

NC, NS, LANES = 2, 16, 16
NW = NC * NS
B_PER_W = BATCH // NW
IDX_PER_W = B_PER_W * N_FIELDS

FEAT = 1000000
PER_SUB = 62496
SCH = 10416
N_SCH = PER_SUB // SCH
REM = FEAT - NS * PER_SUB


def _first_order_sc(idx_f, w_row):
    mesh = plsc.VectorSubcoreMesh(core_axis_name="c", subcore_axis_name="s")

    @functools.partial(
        pl.kernel,
        out_type=jax.ShapeDtypeStruct((1, BATCH), jnp.float32),
        mesh=mesh,
        scratch_types=[
            pltpu.VMEM((IDX_PER_W,), jnp.int32),
            pltpu.VMEM((IDX_PER_W,), jnp.float32),
            pltpu.VMEM((B_PER_W,), jnp.float32),
            pltpu.VMEM((SCH,), jnp.float32),
            pltpu.VMEM((SCH,), jnp.float32),
            pltpu.VMEM_SHARED((FEAT,), jnp.float32),
            pltpu.SemaphoreType.DMA,
            pltpu.SemaphoreType.DMA,
            pltpu.SemaphoreType.DMA,
            pltpu.SemaphoreType.DMA,
            pltpu.SemaphoreType.DMA,
            pltpu.SemaphoreType.DMA,
        ],
    )
    def k(idx_hbm, w_hbm, out_hbm, idx_v, vals_v, out_v, buf_a, buf_b,
          w_sh, sem, la, lb, sa, sb, isem):
        sid = lax.axis_index("s")
        wid = sid * NC + lax.axis_index("c")
        idx_copies = [
            pltpu.make_async_copy(
                idx_hbm.at[pl.ds(f * BATCH + wid * B_PER_W, B_PER_W)],
                idx_v.at[pl.ds(f * B_PER_W, B_PER_W)], isem)
            for f in range(N_FIELDS)
        ]
        for c in idx_copies:
            c.start()

        bufs = (buf_a, buf_b)
        lsems = (la, lb)
        ssems = (sa, sb)
        loads = [
            pltpu.make_async_copy(
                w_hbm.at[0].at[pl.ds(sid * PER_SUB + t * SCH, SCH)],
                bufs[t % 2], lsems[t % 2])
            for t in range(N_SCH)
        ]
        stores = [
            pltpu.make_async_copy(
                bufs[t % 2],
                w_sh.at[pl.ds(sid * PER_SUB + t * SCH, SCH)], ssems[t % 2])
            for t in range(N_SCH)
        ]
        loads[0].start()
        for t in range(N_SCH):
            if t + 1 < N_SCH:
                if t - 1 >= 0:
                    stores[t - 1].wait()
                loads[t + 1].start()
            loads[t].wait()
            stores[t].start()
        stores[N_SCH - 2].wait()
        stores[N_SCH - 1].wait()

        @pl.when(sid == 0)
        def _():
            pltpu.async_copy(w_hbm.at[0].at[pl.ds(NS * PER_SUB, REM)],
                             buf_a.at[pl.ds(0, REM)], la).wait()
            pltpu.async_copy(buf_a.at[pl.ds(0, REM)],
                             w_sh.at[pl.ds(NS * PER_SUB, REM)], sa).wait()

        for c in idx_copies:
            c.wait()
        plsc.subcore_barrier()
        pltpu.async_copy(w_sh.at[idx_v], vals_v, sem).wait()

        def reduce_group(jg, _):
            base = jg * LANES
            acc = vals_v[pl.ds(base, LANES)]
            for f in range(1, N_FIELDS):
                acc = acc + vals_v[pl.ds(f * B_PER_W + base, LANES)]
            out_v[pl.ds(base, LANES)] = acc
            return 0

        lax.fori_loop(0, B_PER_W // LANES, reduce_group, 0, unroll=False)
        pltpu.sync_copy(out_v, out_hbm.at[0].at[pl.ds(wid * B_PER_W, B_PER_W)])

    return k(idx_f, w_row)


def _second_order_tc(et):
    BLK = 2048

    def body(e_ref, out_ref):
        x = e_ref[...]
        s = jnp.sum(x, axis=0)
        t1 = jnp.sum(s * s, axis=0, keepdims=True)
        t2 = jnp.sum(jnp.sum(x * x, axis=0), axis=0, keepdims=True)
        out_ref[...] = 0.5 * (t1 - t2)

    return pl.pallas_call(
        body,
        grid=(BATCH // BLK,),
        in_specs=[pl.BlockSpec((N_FIELDS, EMBED_DIM, BLK), lambda i: (0, 0, i))],
        out_specs=pl.BlockSpec((1, BLK), lambda i: (0, i)),
        out_shape=jax.ShapeDtypeStruct((1, BATCH), jnp.float32),
    )(et)


def _add_tc(a, b):
    def body(a_ref, b_ref, out_ref):
        out_ref[...] = a_ref[...] + b_ref[...]

    return pl.pallas_call(
        body,
        out_shape=jax.ShapeDtypeStruct((1, BATCH), jnp.float32),
    )(a, b)


def kernel(sparse_inputs, embedding_inputs, w):
    idx_f = sparse_inputs.T.reshape(-1)
    first = _first_order_sc(idx_f, w.T)
    second = _second_order_tc(embedding_inputs.transpose(1, 2, 0))
    return _add_tc(first, second).T
